# baseline (speedup 1.0000x reference)
import functools

import jax
import jax.numpy as jnp
from jax import lax
from jax.experimental import pallas as pl
from jax.experimental.pallas import tpu as pltpu
from jax.experimental.pallas import tpu_sc as plsc

NC = 2
NS = 16
NW = NC * NS
CH = 128
D = 128


def _pad_to(n, q):
    return ((n + q - 1) // q) * q


def _relu_inplace(buf):
    def row(i, carry):
        for k in range(D // 16):
            sl = (i, pl.ds(k * 16, 16))
            buf[sl] = jnp.maximum(buf[sl], 0.0)
        return carry
    lax.fori_loop(0, CH, row, 0)


def _zero_inplace(buf):
    z = jnp.zeros((16,), jnp.float32)
    def row(i, carry):
        for k in range(D // 16):
            buf[i, pl.ds(k * 16, 16)] = z
        return carry
    lax.fori_loop(0, CH, row, 0)


def _n_real(total_chunks, wid, cpw):
    return lax.max(0, lax.min(cpw, total_chunks - wid * cpw))


NB = 2


def _sc_edge_conv(emat, src2d, dst2d, table, n_acc, total_chunks, cpw,
                  chunk_row0=0, init=None):
    spt = n_acc // NS
    nz = spt // CH
    have_init = init is not None
    args = (emat, src2d, dst2d, table) + ((init,) if have_init else ())

    @functools.partial(
        pl.kernel,
        out_type=jax.ShapeDtypeStruct((NC, n_acc, D), jnp.float32),
        mesh=plsc.VectorSubcoreMesh(core_axis_name="c", subcore_axis_name="s",
                                    num_cores=NC, num_subcores=NS),
        compiler_params=pltpu.CompilerParams(internal_scratch_in_bytes=65536),
        scratch_types=[
            pltpu.VMEM((16, CH), jnp.int32),
            pltpu.VMEM((16, CH), jnp.int32),
            pltpu.VMEM_SHARED((n_acc, D), jnp.float32),
            [pltpu.SemaphoreType.DMA for _ in range(NB)],
            [pltpu.SemaphoreType.DMA for _ in range(NB)],
        ],
    )
    def k(*refs):
        if have_init:
            (emat_h, src_h, dst_h, x_h, init_h, out_h,
             sidx, didx, acc, gsems, ssems) = refs
        else:
            (emat_h, src_h, dst_h, x_h, out_h,
             sidx, didx, acc, gsems, ssems) = refs
            init_h = None
        cid = lax.axis_index("c")
        sid = lax.axis_index("s")
        wid = cid * NS + sid
        grow = chunk_row0 + wid * cpw
        pltpu.sync_copy(src_h.at[pl.ds(grow, 8)], sidx.at[pl.ds(0, 8)])
        pltpu.sync_copy(dst_h.at[pl.ds(grow, 8)], didx.at[pl.ds(0, 8)])
        n = _n_real(total_chunks, wid, cpw)
        base = wid * cpw * CH

        def scoped(*bufs):
            for z in range(nz):
                r0 = sid * spt + z * CH
                if have_init:
                    pltpu.sync_copy(init_h.at[cid, pl.ds(r0, CH)], bufs[0])
                else:
                    if z == 0:
                        _zero_inplace(bufs[0])
                pltpu.sync_copy(bufs[0], acc.at[pl.ds(r0, CH)])
            plsc.subcore_barrier()

            def lg(j, b):
                pltpu.sync_copy(emat_h.at[pl.ds(base + j * CH, CH)], bufs[b])
                pltpu.async_copy(x_h.at[sidx.at[lax.rem(j, 16)]], bufs[b],
                                 gsems[b], add=True)

            def wait64(sem, b):
                pltpu.make_async_copy(emat_h.at[pl.ds(0, CH)], bufs[b],
                                      sem).wait()

            @pl.when(n >= 1)
            def _():
                lg(0, 0)
            @pl.when(n >= 2)
            def _():
                lg(1, 1)

            def body(j, carry):
                @pl.when((lax.rem(j, 8) == 0) & (j + 8 < n))
                def _():
                    hofs = pl.multiple_of(grow + j + 8, 8)
                    rofs = pl.multiple_of(lax.rem(j + 8, 16), 8)
                    pltpu.sync_copy(src_h.at[pl.ds(hofs, 8)],
                                    sidx.at[pl.ds(rofs, 8)])
                    pltpu.sync_copy(dst_h.at[pl.ds(hofs, 8)],
                                    didx.at[pl.ds(rofs, 8)])
                for b in range(NB):
                    @pl.when(j % NB == b)
                    def _():
                        wait64(gsems[b], b)
                        _relu_inplace(bufs[b])
                        pltpu.async_copy(bufs[b],
                                         acc.at[didx.at[lax.rem(j, 16)]],
                                         ssems[b], add=True)
                        b2 = (b + 2) % NB
                        @pl.when(j + 2 < n)
                        def _():
                            @pl.when(j >= NB - 2)
                            def _():
                                wait64(ssems[b2], b2)
                            lg(j + 2, b2)
                return carry
            lax.fori_loop(0, n, body, 0)
            for t in range(1, NB + 1):
                for b in range(NB):
                    @pl.when((n >= t) & ((n - t) % NB == b))
                    def _():
                        wait64(ssems[b], b)
            plsc.subcore_barrier()
            for z in range(nz):
                r0 = sid * spt + z * CH
                pltpu.sync_copy(acc.at[pl.ds(r0, CH)], bufs[0])
                pltpu.sync_copy(bufs[0], out_h.at[cid, pl.ds(r0, CH)])

        pl.run_scoped(scoped,
                      *[pltpu.VMEM((CH, D), jnp.float32) for _ in range(NB)])

    return k(*args)


def _sc_gather_scatter(table, gidx2d, sidx2d, n_acc, total_chunks, cpw):
    spt = n_acc // NS
    nz = spt // CH

    @functools.partial(
        pl.kernel,
        out_type=jax.ShapeDtypeStruct((NC, n_acc, D), jnp.float32),
        mesh=plsc.VectorSubcoreMesh(core_axis_name="c", subcore_axis_name="s",
                                    num_cores=NC, num_subcores=NS),
        compiler_params=pltpu.CompilerParams(internal_scratch_in_bytes=65536),
        scratch_types=[
            pltpu.VMEM((16, CH), jnp.int32),
            pltpu.VMEM((16, CH), jnp.int32),
            pltpu.VMEM_SHARED((n_acc, D), jnp.float32),
            [pltpu.SemaphoreType.DMA for _ in range(NB)],
            [pltpu.SemaphoreType.DMA for _ in range(NB)],
        ],
    )
    def k(x_h, g_h, s_h, out_h, gidx, sidx, acc, gsems, ssems):
        cid = lax.axis_index("c")
        sid = lax.axis_index("s")
        wid = cid * NS + sid
        start = pl.multiple_of(wid * cpw // 8 * 8, 8)
        off = wid * cpw - start
        pltpu.sync_copy(g_h.at[pl.ds(start, 16)], gidx)
        pltpu.sync_copy(s_h.at[pl.ds(start, 16)], sidx)
        n = _n_real(total_chunks, wid, cpw)

        def scoped(*bufs):
            _zero_inplace(bufs[0])
            for z in range(nz):
                pltpu.sync_copy(bufs[0], acc.at[pl.ds(sid * spt + z * CH, CH)])
            plsc.subcore_barrier()

            def g(j, b):
                pltpu.async_copy(x_h.at[gidx.at[off + j]], bufs[b], gsems[b])

            def wait64(sem, b):
                pltpu.make_async_copy(x_h.at[pl.ds(0, CH)], bufs[b],
                                      sem).wait()

            g(0, 0)
            g(1, 1)

            def body(j, carry):
                for b in range(NB):
                    @pl.when(j % NB == b)
                    def _():
                        wait64(gsems[b], b)
                        pltpu.async_copy(bufs[b], acc.at[sidx.at[off + j]],
                                         ssems[b], add=True)
                        b2 = (b + 2) % NB
                        @pl.when(j + 2 < n)
                        def _():
                            @pl.when(j >= NB - 2)
                            def _():
                                wait64(ssems[b2], b2)
                            g(j + 2, b2)
                return carry
            lax.fori_loop(0, n, body, 0)
            for t in range(1, NB + 1):
                for b in range(NB):
                    @pl.when((n >= t) & ((n - t) % NB == b))
                    def _():
                        wait64(ssems[b], b)
            plsc.subcore_barrier()
            for z in range(nz):
                r0 = sid * spt + z * CH
                pltpu.sync_copy(acc.at[pl.ds(r0, CH)], bufs[0])
                pltpu.sync_copy(bufs[0], out_h.at[cid, pl.ds(r0, CH)])

        pl.run_scoped(scoped,
                      *[pltpu.VMEM((CH, D), jnp.float32) for _ in range(NB)])

    return k(table, gidx2d, sidx2d)


_DOT = functools.partial(
    lax.dot_general,
    dimension_numbers=(((1,), (0,)), ((), ())),
    preferred_element_type=jnp.float32,
)


def _mm_body(a_ref, w_ref, o_ref):
    o_ref[...] = _DOT(a_ref[...], w_ref[...])


def _mm(a, w, blk, row0=0, nrows=None):
    K = a.shape[1]
    if nrows is None:
        nrows = a.shape[0]
    bi = row0 // blk
    return pl.pallas_call(
        _mm_body,
        grid=(nrows // blk,),
        in_specs=[
            pl.BlockSpec((blk, K), lambda i: (i + bi, 0)),
            pl.BlockSpec((K, w.shape[1]), lambda i: (0, 0)),
        ],
        out_specs=pl.BlockSpec((blk, w.shape[1]), lambda i: (i, 0)),
        out_shape=jax.ShapeDtypeStruct((nrows, w.shape[1]), jnp.float32),
    )(a, w)


def _fuse_pre_body(x_ref, p_ref, w_ref, o_ref):
    s = x_ref[...] + p_ref[0] + p_ref[1]
    o_ref[...] = jnp.maximum(_DOT(s, w_ref[...]), 0.0)


def _fuse_post_body(x_ref, p_ref, w_ref, o_ref):
    s = p_ref[0] + p_ref[1]
    o_ref[...] = x_ref[...] + jnp.maximum(_DOT(s, w_ref[...]), 0.0)


def _fuse(body, x, p, w, blk):
    N = x.shape[0]
    spec = pl.BlockSpec((blk, D), lambda i: (i, 0))
    pspec = pl.BlockSpec((NC, blk, D), lambda i: (0, i, 0))
    return pl.pallas_call(
        body,
        grid=(N // blk,),
        in_specs=[spec, pspec, pl.BlockSpec((D, D), lambda i: (0, 0))],
        out_specs=spec,
        out_shape=jax.ShapeDtypeStruct((N, D), jnp.float32),
    )(x, p, w)


def _pad_idx(idx_row, total, n_seg, n_acc):
    i = idx_row.astype(jnp.int32)
    pad = total - i.shape[0]
    if pad:
        if n_acc > n_seg:
            fill = n_seg + jnp.arange(pad, dtype=jnp.int32) % (n_acc - n_seg)
        else:
            fill = jnp.zeros((pad,), jnp.int32)
        i = jnp.concatenate([i, fill])
    return i.reshape(total // CH, CH)


def kernel(x, edge_index, edge_attr, x_clique, node2clique_index,
           clique_edge_index, clique_edge_attr, W_edge, W_nodes, W_n2c,
           W_cedge, W_clique, W_c2n):
    n_nodes = x.shape[0]
    n_cliques = x_clique.shape[0]
    n_edges = edge_index.shape[1]
    n_pairs = node2clique_index.shape[1]
    n_cedges = clique_edge_index.shape[1]

    n_acc_n = _pad_to(n_nodes + 1, NS * CH)
    n_acc_c = _pad_to(n_cliques + 1, NS * CH)

    ep = _pad_to(n_edges, NW * CH * 8)
    cep = _pad_to(n_cedges, NW * CH * 8)
    p_total = (n_pairs + CH - 1) // CH
    p_cpw = (p_total + NW - 1) // NW
    pp = ((NW - 1) * p_cpw // 8 * 8 + 16) * CH

    src = _pad_idx(edge_index[0], ep, 0, 0)
    dst = _pad_idx(edge_index[1], ep, n_nodes, n_acc_n)
    csrc = _pad_idx(clique_edge_index[0], cep, 0, 0)
    cdst = _pad_idx(clique_edge_index[1], cep, n_cliques, n_acc_c)
    nidx = _pad_idx(node2clique_index[0], pp, 0, 0)
    cidx_c = _pad_idx(node2clique_index[1], pp, n_cliques, n_acc_c)
    cidx_g = _pad_idx(node2clique_index[1], pp, 0, 0)
    nidx_s = _pad_idx(node2clique_index[0], pp, n_nodes, n_acc_n)

    ec_total = (n_edges + CH - 1) // CH
    cec_total = (n_cedges + CH - 1) // CH

    split = 1248
    e0 = split * CH
    cpw1 = _pad_to((ec_total - split + NW - 1) // NW, 8)
    emat_a = _mm(edge_attr, W_edge, blk=1024, row0=0, nrows=e0)
    emat_b = _mm(edge_attr, W_edge, blk=512, row0=e0, nrows=n_edges - e0)
    cemat = _mm(clique_edge_attr, W_cedge, blk=2000)

    p1 = _sc_edge_conv(emat_a, src, dst, x, n_acc_n, split, cpw1, 0)
    agg = _sc_edge_conv(emat_b, src, dst, x, n_acc_n, ec_total - split, cpw1,
                        split, init=p1)
    x_n = _fuse(_fuse_pre_body, x, agg, W_nodes, blk=2000)

    g = _sc_gather_scatter(x_n, nidx, cidx_c, n_acc_c, p_total, p_cpw)
    x_c = _fuse(_fuse_post_body, x_clique, g, W_n2c, blk=2000)

    cagg = _sc_edge_conv(cemat, csrc, cdst, x_c, n_acc_c, cec_total,
                         cpw=8)
    x_c2 = _fuse(_fuse_pre_body, x_c, cagg, W_clique, blk=2000)

    h = _sc_gather_scatter(x_c2, cidx_g, nidx_s, n_acc_n, p_total, p_cpw)
    x_out = _fuse(_fuse_post_body, x_n, h, W_c2n, blk=2000)

    return (x_out, x_c2)


# speedup vs baseline: 1.1919x; 1.1919x over previous
import functools

import jax
import jax.numpy as jnp
from jax import lax
from jax.experimental import pallas as pl
from jax.experimental.pallas import tpu as pltpu
from jax.experimental.pallas import tpu_sc as plsc

NC = 2
NS = 16
NW = NC * NS
CH = 128
D = 128


def _pad_to(n, q):
    return ((n + q - 1) // q) * q


def _relu_inplace(buf):
    def row(i, carry):
        for k in range(D // 16):
            sl = (i, pl.ds(k * 16, 16))
            buf[sl] = jnp.maximum(buf[sl], 0.0)
        return carry
    lax.fori_loop(0, CH, row, 0)


def _zero_inplace(buf):
    z = jnp.zeros((16,), jnp.float32)
    def row(i, carry):
        for k in range(D // 16):
            buf[i, pl.ds(k * 16, 16)] = z
        return carry
    lax.fori_loop(0, CH, row, 0)


def _n_real(total_chunks, wid, cpw):
    return lax.max(0, lax.min(cpw, total_chunks - wid * cpw))


NB = 2


def _sc_edge_conv(emat, src2d, dst2d, table, n_acc, total_chunks, cpw,
                  chunk_row0=0, init=None):
    spt = n_acc // NS
    nz = spt // CH
    have_init = init is not None
    args = (emat, src2d, dst2d, table) + ((init,) if have_init else ())

    @functools.partial(
        pl.kernel,
        out_type=jax.ShapeDtypeStruct((NC, n_acc, D), jnp.float32),
        mesh=plsc.VectorSubcoreMesh(core_axis_name="c", subcore_axis_name="s",
                                    num_cores=NC, num_subcores=NS),
        compiler_params=pltpu.CompilerParams(internal_scratch_in_bytes=65536),
        scratch_types=[
            pltpu.VMEM((16, CH), jnp.int32),
            pltpu.VMEM((16, CH), jnp.int32),
            pltpu.VMEM_SHARED((n_acc, D), jnp.float32),
            [pltpu.SemaphoreType.DMA for _ in range(NB)],
            [pltpu.SemaphoreType.DMA for _ in range(NB)],
        ],
    )
    def k(*refs):
        if have_init:
            (emat_h, src_h, dst_h, x_h, init_h, out_h,
             sidx, didx, acc, gsems, ssems) = refs
        else:
            (emat_h, src_h, dst_h, x_h, out_h,
             sidx, didx, acc, gsems, ssems) = refs
            init_h = None
        cid = lax.axis_index("c")
        sid = lax.axis_index("s")
        wid = cid * NS + sid
        grow = chunk_row0 + wid * cpw
        pltpu.sync_copy(src_h.at[pl.ds(grow, 8)], sidx.at[pl.ds(0, 8)])
        pltpu.sync_copy(dst_h.at[pl.ds(grow, 8)], didx.at[pl.ds(0, 8)])
        n = _n_real(total_chunks, wid, cpw)
        base = wid * cpw * CH

        def scoped(*bufs):
            for z in range(nz):
                r0 = sid * spt + z * CH
                if have_init:
                    pltpu.sync_copy(init_h.at[cid, pl.ds(r0, CH)], bufs[0])
                else:
                    if z == 0:
                        _zero_inplace(bufs[0])
                pltpu.sync_copy(bufs[0], acc.at[pl.ds(r0, CH)])
            plsc.subcore_barrier()

            def lg(j, b):
                pltpu.sync_copy(emat_h.at[pl.ds(base + j * CH, CH)], bufs[b])
                pltpu.async_copy(x_h.at[sidx.at[lax.rem(j, 16)]], bufs[b],
                                 gsems[b], add=True)

            def wait64(sem, b):
                pltpu.make_async_copy(emat_h.at[pl.ds(0, CH)], bufs[b],
                                      sem).wait()

            @pl.when(n >= 1)
            def _():
                lg(0, 0)
            @pl.when(n >= 2)
            def _():
                lg(1, 1)

            def body(j, carry):
                @pl.when((lax.rem(j, 8) == 0) & (j + 8 < n))
                def _():
                    hofs = pl.multiple_of(grow + j + 8, 8)
                    rofs = pl.multiple_of(lax.rem(j + 8, 16), 8)
                    pltpu.sync_copy(src_h.at[pl.ds(hofs, 8)],
                                    sidx.at[pl.ds(rofs, 8)])
                    pltpu.sync_copy(dst_h.at[pl.ds(hofs, 8)],
                                    didx.at[pl.ds(rofs, 8)])
                for b in range(NB):
                    @pl.when(j % NB == b)
                    def _():
                        wait64(gsems[b], b)
                        _relu_inplace(bufs[b])
                        pltpu.async_copy(bufs[b],
                                         acc.at[didx.at[lax.rem(j, 16)]],
                                         ssems[b], add=True)
                        b2 = (b + 2) % NB
                        @pl.when(j + 2 < n)
                        def _():
                            @pl.when(j >= NB - 2)
                            def _():
                                wait64(ssems[b2], b2)
                            lg(j + 2, b2)
                return carry
            lax.fori_loop(0, n, body, 0)
            for t in range(1, NB + 1):
                for b in range(NB):
                    @pl.when((n >= t) & ((n - t) % NB == b))
                    def _():
                        wait64(ssems[b], b)
            plsc.subcore_barrier()
            for z in range(nz):
                r0 = sid * spt + z * CH
                pltpu.sync_copy(acc.at[pl.ds(r0, CH)], bufs[0])
                pltpu.sync_copy(bufs[0], out_h.at[cid, pl.ds(r0, CH)])

        pl.run_scoped(scoped,
                      *[pltpu.VMEM((CH, D), jnp.float32) for _ in range(NB)])

    return k(*args)


def _sc_gather_scatter(table, gidx2d, sidx2d, n_acc, total_chunks, cpw):
    spt = n_acc // NS
    nz = spt // CH

    @functools.partial(
        pl.kernel,
        out_type=jax.ShapeDtypeStruct((NC, n_acc, D), jnp.float32),
        mesh=plsc.VectorSubcoreMesh(core_axis_name="c", subcore_axis_name="s",
                                    num_cores=NC, num_subcores=NS),
        compiler_params=pltpu.CompilerParams(internal_scratch_in_bytes=65536),
        scratch_types=[
            pltpu.VMEM((16, CH), jnp.int32),
            pltpu.VMEM((16, CH), jnp.int32),
            pltpu.VMEM_SHARED((n_acc, D), jnp.float32),
            [pltpu.SemaphoreType.DMA for _ in range(NB)],
            [pltpu.SemaphoreType.DMA for _ in range(NB)],
        ],
    )
    def k(x_h, g_h, s_h, out_h, gidx, sidx, acc, gsems, ssems):
        cid = lax.axis_index("c")
        sid = lax.axis_index("s")
        wid = cid * NS + sid
        start = pl.multiple_of(wid * cpw // 8 * 8, 8)
        off = wid * cpw - start
        pltpu.sync_copy(g_h.at[pl.ds(start, 16)], gidx)
        pltpu.sync_copy(s_h.at[pl.ds(start, 16)], sidx)
        n = _n_real(total_chunks, wid, cpw)

        def scoped(*bufs):
            _zero_inplace(bufs[0])
            for z in range(nz):
                pltpu.sync_copy(bufs[0], acc.at[pl.ds(sid * spt + z * CH, CH)])
            plsc.subcore_barrier()

            def g(j, b):
                pltpu.async_copy(x_h.at[gidx.at[off + j]], bufs[b], gsems[b])

            def wait64(sem, b):
                pltpu.make_async_copy(x_h.at[pl.ds(0, CH)], bufs[b],
                                      sem).wait()

            g(0, 0)
            g(1, 1)

            def body(j, carry):
                for b in range(NB):
                    @pl.when(j % NB == b)
                    def _():
                        wait64(gsems[b], b)
                        pltpu.async_copy(bufs[b], acc.at[sidx.at[off + j]],
                                         ssems[b], add=True)
                        b2 = (b + 2) % NB
                        @pl.when(j + 2 < n)
                        def _():
                            @pl.when(j >= NB - 2)
                            def _():
                                wait64(ssems[b2], b2)
                            g(j + 2, b2)
                return carry
            lax.fori_loop(0, n, body, 0)
            for t in range(1, NB + 1):
                for b in range(NB):
                    @pl.when((n >= t) & ((n - t) % NB == b))
                    def _():
                        wait64(ssems[b], b)
            plsc.subcore_barrier()
            for z in range(nz):
                r0 = sid * spt + z * CH
                pltpu.sync_copy(acc.at[pl.ds(r0, CH)], bufs[0])
                pltpu.sync_copy(bufs[0], out_h.at[cid, pl.ds(r0, CH)])

        pl.run_scoped(scoped,
                      *[pltpu.VMEM((CH, D), jnp.float32) for _ in range(NB)])

    return k(table, gidx2d, sidx2d)


_DOT = functools.partial(
    lax.dot_general,
    dimension_numbers=(((1,), (0,)), ((), ())),
    preferred_element_type=jnp.float32,
)


def _mm_body(a_ref, w_ref, o_ref):
    o_ref[...] = _DOT(a_ref[...], w_ref[...])


def _mm(a, w, blk, row0=0, nrows=None):
    K = a.shape[1]
    if nrows is None:
        nrows = a.shape[0]
    bi = row0 // blk
    return pl.pallas_call(
        _mm_body,
        grid=(pl.cdiv(nrows, blk),),
        in_specs=[
            pl.BlockSpec((blk, K), lambda i: (i + bi, 0)),
            pl.BlockSpec((K, w.shape[1]), lambda i: (0, 0)),
        ],
        out_specs=pl.BlockSpec((blk, w.shape[1]), lambda i: (i, 0)),
        out_shape=jax.ShapeDtypeStruct((nrows, w.shape[1]), jnp.float32),
    )(a, w)


def _fuse_pre_body(x_ref, p_ref, w_ref, o_ref):
    s = x_ref[...] + p_ref[0] + p_ref[1]
    o_ref[...] = jnp.maximum(_DOT(s, w_ref[...]), 0.0)


def _fuse_post_body(x_ref, p_ref, w_ref, o_ref):
    s = p_ref[0] + p_ref[1]
    o_ref[...] = x_ref[...] + jnp.maximum(_DOT(s, w_ref[...]), 0.0)


def _fuse(body, x, p, w, blk):
    N = x.shape[0]
    spec = pl.BlockSpec((blk, D), lambda i: (i, 0))
    pspec = pl.BlockSpec((NC, blk, D), lambda i: (0, i, 0))
    return pl.pallas_call(
        body,
        grid=(N // blk,),
        in_specs=[spec, pspec, pl.BlockSpec((D, D), lambda i: (0, 0))],
        out_specs=spec,
        out_shape=jax.ShapeDtypeStruct((N, D), jnp.float32),
    )(x, p, w)


def _pad_idx(idx_row, total, n_seg, n_acc):
    i = idx_row.astype(jnp.int32)
    pad = total - i.shape[0]
    if pad:
        if n_acc > n_seg:
            fill = n_seg + jnp.arange(pad, dtype=jnp.int32) % (n_acc - n_seg)
        else:
            fill = jnp.zeros((pad,), jnp.int32)
        i = jnp.concatenate([i, fill])
    return i.reshape(total // CH, CH)


def kernel(x, edge_index, edge_attr, x_clique, node2clique_index,
           clique_edge_index, clique_edge_attr, W_edge, W_nodes, W_n2c,
           W_cedge, W_clique, W_c2n):
    n_nodes = x.shape[0]
    n_cliques = x_clique.shape[0]
    n_edges = edge_index.shape[1]
    n_pairs = node2clique_index.shape[1]
    n_cedges = clique_edge_index.shape[1]

    n_acc_n = _pad_to(n_nodes + 1, NS * CH)
    n_acc_c = _pad_to(n_cliques + 1, NS * CH)

    ep = _pad_to(n_edges, NW * CH * 8)
    cep = _pad_to(n_cedges, NW * CH * 8)
    p_total = (n_pairs + CH - 1) // CH
    p_cpw = (p_total + NW - 1) // NW
    pp = ((NW - 1) * p_cpw // 8 * 8 + 16) * CH

    src = _pad_idx(edge_index[0], ep, 0, 0)
    dst = _pad_idx(edge_index[1], ep, n_nodes, n_acc_n)
    csrc = _pad_idx(clique_edge_index[0], cep, 0, 0)
    cdst = _pad_idx(clique_edge_index[1], cep, n_cliques, n_acc_c)
    nidx = _pad_idx(node2clique_index[0], pp, 0, 0)
    cidx_c = _pad_idx(node2clique_index[1], pp, n_cliques, n_acc_c)
    cidx_g = _pad_idx(node2clique_index[1], pp, 0, 0)
    nidx_s = _pad_idx(node2clique_index[0], pp, n_nodes, n_acc_n)

    ec_total = (n_edges + CH - 1) // CH
    cec_total = (n_cedges + CH - 1) // CH

    split = 1248
    e0 = split * CH
    cpw1 = _pad_to((ec_total - split + NW - 1) // NW, 8)
    emat_a = _mm(edge_attr, W_edge, blk=2048, row0=0, nrows=e0)
    emat_b = _mm(edge_attr, W_edge, blk=2048, row0=e0, nrows=n_edges - e0)
    cemat = _mm(clique_edge_attr, W_cedge, blk=2000)

    p1 = _sc_edge_conv(emat_a, src, dst, x, n_acc_n, split, cpw1, 0)
    agg = _sc_edge_conv(emat_b, src, dst, x, n_acc_n, ec_total - split, cpw1,
                        split, init=p1)
    x_n = _fuse(_fuse_pre_body, x, agg, W_nodes, blk=2000)

    g = _sc_gather_scatter(x_n, nidx, cidx_c, n_acc_c, p_total, p_cpw)
    x_c = _fuse(_fuse_post_body, x_clique, g, W_n2c, blk=2000)

    cagg = _sc_edge_conv(cemat, csrc, cdst, x_c, n_acc_c, cec_total,
                         cpw=8)
    x_c2 = _fuse(_fuse_pre_body, x_c, cagg, W_clique, blk=2000)

    h = _sc_gather_scatter(x_c2, cidx_g, nidx_s, n_acc_n, p_total, p_cpw)
    x_out = _fuse(_fuse_post_body, x_n, h, W_c2n, blk=2000)

    return (x_out, x_c2)


# speedup vs baseline: 1.2209x; 1.0243x over previous
import functools

import jax
import jax.numpy as jnp
from jax import lax
from jax.experimental import pallas as pl
from jax.experimental.pallas import tpu as pltpu
from jax.experimental.pallas import tpu_sc as plsc

NC = 2
NS = 16
NW = NC * NS
CH = 128
D = 128


def _pad_to(n, q):
    return ((n + q - 1) // q) * q


def _relu_inplace(buf):
    def row(i, carry):
        for k in range(D // 16):
            sl = (i, pl.ds(k * 16, 16))
            buf[sl] = jnp.maximum(buf[sl], 0.0)
        return carry
    lax.fori_loop(0, CH, row, 0)


def _zero_inplace(buf):
    z = jnp.zeros((16,), jnp.float32)
    def row(i, carry):
        for k in range(D // 16):
            buf[i, pl.ds(k * 16, 16)] = z
        return carry
    lax.fori_loop(0, CH, row, 0)


def _n_real(total_chunks, wid, cpw):
    return lax.max(0, lax.min(cpw, total_chunks - wid * cpw))


NB = 2


def _sc_edge_conv(emat, eidx3d, table, n_acc, total_chunks, cpw,
                  chunk_row0=0, init=None):
    spt = n_acc // NS
    nz = spt // CH
    have_init = init is not None
    args = (emat, eidx3d, table) + ((init,) if have_init else ())

    @functools.partial(
        pl.kernel,
        out_type=jax.ShapeDtypeStruct((NC, n_acc, D), jnp.float32),
        mesh=plsc.VectorSubcoreMesh(core_axis_name="c", subcore_axis_name="s",
                                    num_cores=NC, num_subcores=NS),
        compiler_params=pltpu.CompilerParams(internal_scratch_in_bytes=65536),
        scratch_types=[
            pltpu.VMEM((16, CH), jnp.int32),
            pltpu.VMEM((16, CH), jnp.int32),
            pltpu.VMEM_SHARED((n_acc, D), jnp.float32),
            [pltpu.SemaphoreType.DMA for _ in range(NB)],
            [pltpu.SemaphoreType.DMA for _ in range(NB)],
        ],
    )
    def k(*refs):
        if have_init:
            (emat_h, eidx_h, x_h, init_h, out_h,
             sidx, didx, acc, gsems, ssems) = refs
        else:
            (emat_h, eidx_h, x_h, out_h,
             sidx, didx, acc, gsems, ssems) = refs
            init_h = None
        cid = lax.axis_index("c")
        sid = lax.axis_index("s")
        wid = cid * NS + sid
        grow = chunk_row0 + wid * cpw
        pltpu.sync_copy(eidx_h.at[0, pl.ds(grow, 8)], sidx.at[pl.ds(0, 8)])
        pltpu.sync_copy(eidx_h.at[1, pl.ds(grow, 8)], didx.at[pl.ds(0, 8)])
        n = _n_real(total_chunks, wid, cpw)
        base = wid * cpw * CH

        def scoped(*bufs):
            for z in range(nz):
                r0 = sid * spt + z * CH
                if have_init:
                    pltpu.sync_copy(init_h.at[cid, pl.ds(r0, CH)], bufs[0])
                else:
                    if z == 0:
                        _zero_inplace(bufs[0])
                pltpu.sync_copy(bufs[0], acc.at[pl.ds(r0, CH)])
            plsc.subcore_barrier()

            def lg(j, b):
                pltpu.sync_copy(emat_h.at[pl.ds(base + j * CH, CH)], bufs[b])
                pltpu.async_copy(x_h.at[sidx.at[lax.rem(j, 16)]], bufs[b],
                                 gsems[b], add=True)

            def wait64(sem, b):
                pltpu.make_async_copy(emat_h.at[pl.ds(0, CH)], bufs[b],
                                      sem).wait()

            @pl.when(n >= 1)
            def _():
                lg(0, 0)
            @pl.when(n >= 2)
            def _():
                lg(1, 1)

            def body(j, carry):
                @pl.when((lax.rem(j, 8) == 0) & (j + 8 < n))
                def _():
                    hofs = pl.multiple_of(grow + j + 8, 8)
                    rofs = pl.multiple_of(lax.rem(j + 8, 16), 8)
                    pltpu.sync_copy(eidx_h.at[0, pl.ds(hofs, 8)],
                                    sidx.at[pl.ds(rofs, 8)])
                    pltpu.sync_copy(eidx_h.at[1, pl.ds(hofs, 8)],
                                    didx.at[pl.ds(rofs, 8)])
                for b in range(NB):
                    @pl.when(j % NB == b)
                    def _():
                        wait64(gsems[b], b)
                        _relu_inplace(bufs[b])
                        pltpu.async_copy(bufs[b],
                                         acc.at[didx.at[lax.rem(j, 16)]],
                                         ssems[b], add=True)
                        b2 = (b + 2) % NB
                        @pl.when(j + 2 < n)
                        def _():
                            @pl.when(j >= NB - 2)
                            def _():
                                wait64(ssems[b2], b2)
                            lg(j + 2, b2)
                return carry
            lax.fori_loop(0, n, body, 0)
            for t in range(1, NB + 1):
                for b in range(NB):
                    @pl.when((n >= t) & ((n - t) % NB == b))
                    def _():
                        wait64(ssems[b], b)
            plsc.subcore_barrier()
            for z in range(nz):
                r0 = sid * spt + z * CH
                pltpu.sync_copy(acc.at[pl.ds(r0, CH)], bufs[0])
                pltpu.sync_copy(bufs[0], out_h.at[cid, pl.ds(r0, CH)])

        pl.run_scoped(scoped,
                      *[pltpu.VMEM((CH, D), jnp.float32) for _ in range(NB)])

    return k(*args)


def _sc_gather_scatter(table, gidx2d, sidx2d, n_acc, total_chunks, cpw):
    spt = n_acc // NS
    nz = spt // CH

    @functools.partial(
        pl.kernel,
        out_type=jax.ShapeDtypeStruct((NC, n_acc, D), jnp.float32),
        mesh=plsc.VectorSubcoreMesh(core_axis_name="c", subcore_axis_name="s",
                                    num_cores=NC, num_subcores=NS),
        compiler_params=pltpu.CompilerParams(internal_scratch_in_bytes=65536),
        scratch_types=[
            pltpu.VMEM((16, CH), jnp.int32),
            pltpu.VMEM((16, CH), jnp.int32),
            pltpu.VMEM_SHARED((n_acc, D), jnp.float32),
            [pltpu.SemaphoreType.DMA for _ in range(NB)],
            [pltpu.SemaphoreType.DMA for _ in range(NB)],
        ],
    )
    def k(x_h, g_h, s_h, out_h, gidx, sidx, acc, gsems, ssems):
        cid = lax.axis_index("c")
        sid = lax.axis_index("s")
        wid = cid * NS + sid
        start = pl.multiple_of(wid * cpw // 8 * 8, 8)
        off = wid * cpw - start
        pltpu.sync_copy(g_h.at[pl.ds(start, 16)], gidx)
        pltpu.sync_copy(s_h.at[pl.ds(start, 16)], sidx)
        n = _n_real(total_chunks, wid, cpw)

        def scoped(*bufs):
            _zero_inplace(bufs[0])
            for z in range(nz):
                pltpu.sync_copy(bufs[0], acc.at[pl.ds(sid * spt + z * CH, CH)])
            plsc.subcore_barrier()

            def g(j, b):
                pltpu.async_copy(x_h.at[gidx.at[off + j]], bufs[b], gsems[b])

            def wait64(sem, b):
                pltpu.make_async_copy(x_h.at[pl.ds(0, CH)], bufs[b],
                                      sem).wait()

            g(0, 0)
            g(1, 1)

            def body(j, carry):
                for b in range(NB):
                    @pl.when(j % NB == b)
                    def _():
                        wait64(gsems[b], b)
                        pltpu.async_copy(bufs[b], acc.at[sidx.at[off + j]],
                                         ssems[b], add=True)
                        b2 = (b + 2) % NB
                        @pl.when(j + 2 < n)
                        def _():
                            @pl.when(j >= NB - 2)
                            def _():
                                wait64(ssems[b2], b2)
                            g(j + 2, b2)
                return carry
            lax.fori_loop(0, n, body, 0)
            for t in range(1, NB + 1):
                for b in range(NB):
                    @pl.when((n >= t) & ((n - t) % NB == b))
                    def _():
                        wait64(ssems[b], b)
            plsc.subcore_barrier()
            for z in range(nz):
                r0 = sid * spt + z * CH
                pltpu.sync_copy(acc.at[pl.ds(r0, CH)], bufs[0])
                pltpu.sync_copy(bufs[0], out_h.at[cid, pl.ds(r0, CH)])

        pl.run_scoped(scoped,
                      *[pltpu.VMEM((CH, D), jnp.float32) for _ in range(NB)])

    return k(table, gidx2d, sidx2d)


_DOT = functools.partial(
    lax.dot_general,
    dimension_numbers=(((1,), (0,)), ((), ())),
    preferred_element_type=jnp.float32,
)


def _mm_body(a_ref, w_ref, o_ref):
    o_ref[...] = _DOT(a_ref[...], w_ref[...])


def _mm(a, w, blk, row0=0, nrows=None):
    K = a.shape[1]
    if nrows is None:
        nrows = a.shape[0]
    bi = row0 // blk
    return pl.pallas_call(
        _mm_body,
        grid=(pl.cdiv(nrows, blk),),
        in_specs=[
            pl.BlockSpec((blk, K), lambda i: (i + bi, 0)),
            pl.BlockSpec((K, w.shape[1]), lambda i: (0, 0)),
        ],
        out_specs=pl.BlockSpec((blk, w.shape[1]), lambda i: (i, 0)),
        out_shape=jax.ShapeDtypeStruct((nrows, w.shape[1]), jnp.float32),
    )(a, w)


def _fuse_pre_body(x_ref, p_ref, w_ref, o_ref):
    s = x_ref[...] + p_ref[0] + p_ref[1]
    o_ref[...] = jnp.maximum(_DOT(s, w_ref[...]), 0.0)


def _fuse_post_body(x_ref, p_ref, w_ref, o_ref):
    s = p_ref[0] + p_ref[1]
    o_ref[...] = x_ref[...] + jnp.maximum(_DOT(s, w_ref[...]), 0.0)


def _fuse(body, x, p, w, blk):
    N = x.shape[0]
    spec = pl.BlockSpec((blk, D), lambda i: (i, 0))
    pspec = pl.BlockSpec((NC, blk, D), lambda i: (0, i, 0))
    return pl.pallas_call(
        body,
        grid=(N // blk,),
        in_specs=[spec, pspec, pl.BlockSpec((D, D), lambda i: (0, 0))],
        out_specs=spec,
        out_shape=jax.ShapeDtypeStruct((N, D), jnp.float32),
    )(x, p, w)


def _pad_idx(idx_row, total, n_seg, n_acc):
    i = idx_row.astype(jnp.int32)
    pad = total - i.shape[0]
    if pad:
        if n_acc > n_seg:
            fill = n_seg + jnp.arange(pad, dtype=jnp.int32) % (n_acc - n_seg)
        else:
            fill = jnp.zeros((pad,), jnp.int32)
        i = jnp.concatenate([i, fill])
    return i.reshape(total // CH, CH)


def kernel(x, edge_index, edge_attr, x_clique, node2clique_index,
           clique_edge_index, clique_edge_attr, W_edge, W_nodes, W_n2c,
           W_cedge, W_clique, W_c2n):
    n_nodes = x.shape[0]
    n_cliques = x_clique.shape[0]
    n_edges = edge_index.shape[1]
    n_pairs = node2clique_index.shape[1]
    n_cedges = clique_edge_index.shape[1]

    n_acc_n = _pad_to(n_nodes + 1, NS * CH)
    n_acc_c = _pad_to(n_cliques + 1, NS * CH)

    ep = _pad_to(n_edges, NW * CH * 8)
    cep = _pad_to(n_cedges, NW * CH * 8)
    p_total = (n_pairs + CH - 1) // CH
    p_cpw = (p_total + NW - 1) // NW
    pp = ((NW - 1) * p_cpw // 8 * 8 + 16) * CH

    eidx = jnp.pad(edge_index.astype(jnp.int32),
                   ((0, 0), (0, ep - n_edges))).reshape(2, ep // CH, CH)
    ceidx = jnp.pad(clique_edge_index.astype(jnp.int32),
                    ((0, 0), (0, cep - n_cedges))).reshape(2, cep // CH, CH)
    nidx = _pad_idx(node2clique_index[0], pp, 0, 0)
    cidx_c = _pad_idx(node2clique_index[1], pp, n_cliques, n_acc_c)
    cidx_g = _pad_idx(node2clique_index[1], pp, 0, 0)
    nidx_s = _pad_idx(node2clique_index[0], pp, n_nodes, n_acc_n)

    ec_total = (n_edges + CH - 1) // CH
    cec_total = (n_cedges + CH - 1) // CH

    split = 1248
    e0 = split * CH
    cpw1 = _pad_to((ec_total - split + NW - 1) // NW, 8)
    emat_a = _mm(edge_attr, W_edge, blk=2048, row0=0, nrows=e0)
    emat_b = _mm(edge_attr, W_edge, blk=2048, row0=e0, nrows=n_edges - e0)
    cemat = _mm(clique_edge_attr, W_cedge, blk=2000)

    p1 = _sc_edge_conv(emat_a, eidx, x, n_acc_n, split, cpw1, 0)
    agg = _sc_edge_conv(emat_b, eidx, x, n_acc_n, ec_total - split, cpw1,
                        split, init=p1)
    x_n = _fuse(_fuse_pre_body, x, agg, W_nodes, blk=2000)

    g = _sc_gather_scatter(x_n, nidx, cidx_c, n_acc_c, p_total, p_cpw)
    x_c = _fuse(_fuse_post_body, x_clique, g, W_n2c, blk=2000)

    cagg = _sc_edge_conv(cemat, ceidx, x_c, n_acc_c, cec_total,
                         cpw=8)
    x_c2 = _fuse(_fuse_pre_body, x_c, cagg, W_clique, blk=2000)

    h = _sc_gather_scatter(x_c2, cidx_g, nidx_s, n_acc_n, p_total, p_cpw)
    x_out = _fuse(_fuse_post_body, x_n, h, W_c2n, blk=2000)

    return (x_out, x_c2)
